# baseline (speedup 1.0000x reference)
import functools

import jax
import jax.numpy as jnp
from jax import lax
from jax.experimental import pallas as pl
from jax.experimental.pallas import tpu as pltpu
from jax.experimental.pallas import tpu_sc as plsc

B = 4
V = 1024
D = 16
N = 256
L = 5
NW = 32
NELEM = B * N * N
PLANE = N * N
EPT = NELEM // NW
TPB = NW // B
TABW = L * V
CH = EPT // 2

_SCALE = 1.0 / (5.0 + 1e-9)


def _table_body(vec_ref, emb_ref, out_ref):
    vec = vec_ref[...]
    for b in range(B):
        emb = emb_ref[b]
        for j in range(V // 128):
            out_ref[b, :, j, :] = lax.dot_general(
                vec, emb[:, j * 128:(j + 1) * 128], (((1,), (0,)), ((), ())),
                preferred_element_type=jnp.float32) * _SCALE


def _make_table(vec, emb_t):
    return pl.pallas_call(
        _table_body,
        in_specs=[
            pl.BlockSpec((L, D), lambda: (0, 0)),
            pl.BlockSpec((B, D, V), lambda: (0, 0, 0)),
        ],
        out_specs=pl.BlockSpec((B, L, V // 128, 128), lambda: (0, 0, 0, 0)),
        out_shape=jax.ShapeDtypeStruct((B, L, V // 128, 128), jnp.float32),
    )(vec, emb_t)


@functools.partial(
    pl.kernel,
    out_type=jax.ShapeDtypeStruct((NELEM,), jnp.float32),
    mesh=plsc.VectorSubcoreMesh(core_axis_name="c", subcore_axis_name="s"),
    compiler_params=pltpu.CompilerParams(needs_layout_passes=False),
    scratch_types=[
        pltpu.VMEM((2 * L * CH,), jnp.int32),
        pltpu.VMEM((TABW,), jnp.float32),
        pltpu.VMEM((EPT,), jnp.float32),
        pltpu.SemaphoreType.DMA,
        pltpu.SemaphoreType.DMA,
        pltpu.SemaphoreType.DMA,
    ],
)
def _sc_gather(idx_hbm, tab_hbm, out_hbm, idx_v, tab_v, out_v, s0, s1, st):
    wid = lax.axis_index("s") * 2 + lax.axis_index("c")
    b = wid // TPB
    chunk = wid % TPB

    def fire(half, sem):
        return [
            pltpu.async_copy(
                idx_hbm.at[pl.ds(
                    (b * L + l) * PLANE + chunk * EPT + half * CH, CH)],
                idx_v.at[pl.ds((half * L + l) * CH, CH)], sem)
            for l in range(L)
        ]

    c0 = fire(0, s0)
    ct = pltpu.async_copy(tab_hbm.at[pl.ds(b * TABW, TABW)], tab_v, st)
    c1 = fire(1, s1)
    for c in c0:
        c.wait()
    ct.wait()

    def compute_half(half):
        base = half * L * CH
        obase = half * CH

        @plsc.parallel_loop(0, CH // 16, unroll=16)
        def _(i):
            g = [
                plsc.load_gather(
                    tab_v.at[pl.ds(l * V, V)],
                    [idx_v[pl.ds(base + l * CH + i * 16, 16)]])
                for l in range(L)
            ]
            out_v[pl.ds(obase + i * 16, 16)] = (
                (g[0] + g[1]) + (g[2] + g[3]) + g[4])

        return pltpu.async_copy(
            out_v.at[pl.ds(obase, CH)],
            out_hbm.at[pl.ds(wid * EPT + obase, CH)], st)

    o0 = compute_half(0)
    for c in c1:
        c.wait()
    o1 = compute_half(1)
    o0.wait()
    o1.wait()


def kernel(edge_embedding, edge_paths, edge_vector):
    emb_t = edge_embedding.transpose(0, 2, 1)
    tab = _make_table(edge_vector.astype(jnp.float32), emb_t)
    idx_t = (edge_paths.astype(jnp.int32)
             .reshape(B, 32, 8, 2, 128, L)
             .transpose(0, 5, 1, 3, 2, 4)
             .reshape(-1))
    out = _sc_gather(idx_t, tab.reshape(-1))
    return (out.reshape(B, 32, 2, 8, 128)
            .transpose(0, 1, 3, 2, 4)
            .reshape(B, N, N))


# speedup vs baseline: 1.0579x; 1.0579x over previous
import functools

import jax
import jax.numpy as jnp
from jax import lax
from jax.experimental import pallas as pl
from jax.experimental.pallas import tpu as pltpu
from jax.experimental.pallas import tpu_sc as plsc

B = 4
V = 1024
D = 16
N = 256
L = 5
NW = 32
NELEM = B * N * N
PLANE = N * N
EPT = NELEM // NW
TPB = NW // B
TABW = L * V
CH = EPT // 2

_SCALE = 1.0 / (5.0 + 1e-9)


def _table_body(vec_ref, emb_ref, out_ref):
    vec = vec_ref[...]
    for b in range(B):
        emb = emb_ref[b]
        for j in range(V // 128):
            out_ref[b, :, j, :] = lax.dot_general(
                vec, emb[:, j * 128:(j + 1) * 128], (((1,), (0,)), ((), ())),
                preferred_element_type=jnp.float32) * _SCALE


def _make_table(vec, emb_t):
    return pl.pallas_call(
        _table_body,
        in_specs=[
            pl.BlockSpec((L, D), lambda: (0, 0)),
            pl.BlockSpec((B, D, V), lambda: (0, 0, 0)),
        ],
        out_specs=pl.BlockSpec((B, L, V // 128, 128), lambda: (0, 0, 0, 0)),
        out_shape=jax.ShapeDtypeStruct((B, L, V // 128, 128), jnp.float32),
    )(vec, emb_t)


@functools.partial(
    pl.kernel,
    out_type=jax.ShapeDtypeStruct((NELEM,), jnp.float32),
    mesh=plsc.VectorSubcoreMesh(core_axis_name="c", subcore_axis_name="s"),
    compiler_params=pltpu.CompilerParams(needs_layout_passes=False),
    scratch_types=[
        pltpu.VMEM((2 * L * CH,), jnp.int32),
        pltpu.VMEM((TABW,), jnp.float32),
        pltpu.VMEM((EPT,), jnp.float32),
        pltpu.SemaphoreType.DMA,
        pltpu.SemaphoreType.DMA,
        pltpu.SemaphoreType.DMA,
    ],
)
def _sc_gather(idx_hbm, tab_hbm, out_hbm, idx_v, tab_v, out_v, s0, s1, st):
    wid = lax.axis_index("s") * 2 + lax.axis_index("c")
    b = wid // TPB
    chunk = wid % TPB

    def fire(half, sem):
        return [
            pltpu.async_copy(
                idx_hbm.at[pl.ds(
                    (b * L + l) * PLANE + chunk * EPT + half * CH, CH)],
                idx_v.at[pl.ds((half * L + l) * CH, CH)], sem)
            for l in range(L)
        ]

    ct = pltpu.async_copy(tab_hbm.at[pl.ds(b * TABW, TABW)], tab_v, st)
    c0 = fire(0, s0)
    c1 = fire(1, s1)
    for c in c0:
        c.wait()
    ct.wait()

    def compute_half(half):
        base = half * L * CH
        obase = half * CH

        @plsc.parallel_loop(0, CH // 16, unroll=4)
        def _(i):
            g = [
                plsc.load_gather(
                    tab_v.at[pl.ds(l * V, V)],
                    [idx_v[pl.ds(base + l * CH + i * 16, 16)]])
                for l in range(L)
            ]
            out_v[pl.ds(obase + i * 16, 16)] = (
                (g[0] + g[1]) + (g[2] + g[3]) + g[4])

        return pltpu.async_copy(
            out_v.at[pl.ds(obase, CH)],
            out_hbm.at[pl.ds(wid * EPT + obase, CH)], st)

    o0 = compute_half(0)
    for c in c1:
        c.wait()
    o1 = compute_half(1)
    o0.wait()
    o1.wait()


def kernel(edge_embedding, edge_paths, edge_vector):
    emb_t = edge_embedding.transpose(0, 2, 1)
    tab = _make_table(edge_vector.astype(jnp.float32), emb_t)
    idx_t = (edge_paths.astype(jnp.int32)
             .reshape(B, 32, 8, 2, 128, L)
             .transpose(0, 5, 1, 3, 2, 4)
             .reshape(-1))
    out = _sc_gather(idx_t, tab.reshape(-1))
    return (out.reshape(B, 32, 2, 8, 128)
            .transpose(0, 1, 3, 2, 4)
            .reshape(B, N, N))


# speedup vs baseline: 1.0594x; 1.0014x over previous
import functools

import jax
import jax.numpy as jnp
from jax import lax
from jax.experimental import pallas as pl
from jax.experimental.pallas import tpu as pltpu
from jax.experimental.pallas import tpu_sc as plsc

B = 4
V = 1024
D = 16
N = 256
L = 5
NW = 32
NELEM = B * N * N
PLANE = N * N
EPT = NELEM // NW
TPB = NW // B
TABW = L * V
CH = EPT // 2

_SCALE = 1.0 / (5.0 + 1e-9)


def _table_body(vec_ref, emb_ref, out_ref):
    vec = vec_ref[...]
    for b in range(B):
        emb = emb_ref[b]
        for j in range(V // 128):
            out_ref[b, :, j, :] = lax.dot_general(
                vec, emb[:, j * 128:(j + 1) * 128], (((1,), (0,)), ((), ())),
                preferred_element_type=jnp.float32) * _SCALE


def _make_table(vec, emb_t):
    return pl.pallas_call(
        _table_body,
        in_specs=[
            pl.BlockSpec((L, D), lambda: (0, 0)),
            pl.BlockSpec((B, D, V), lambda: (0, 0, 0)),
        ],
        out_specs=pl.BlockSpec((B, L, V // 128, 128), lambda: (0, 0, 0, 0)),
        out_shape=jax.ShapeDtypeStruct((B, L, V // 128, 128), jnp.float32),
    )(vec, emb_t)


@functools.partial(
    pl.kernel,
    out_type=jax.ShapeDtypeStruct((NELEM,), jnp.float32),
    mesh=plsc.VectorSubcoreMesh(core_axis_name="c", subcore_axis_name="s"),
    compiler_params=pltpu.CompilerParams(needs_layout_passes=False),
    scratch_types=[
        pltpu.VMEM((2 * L * CH,), jnp.int32),
        pltpu.VMEM((TABW,), jnp.float32),
        pltpu.VMEM((EPT,), jnp.float32),
        pltpu.SemaphoreType.DMA,
        pltpu.SemaphoreType.DMA,
        pltpu.SemaphoreType.DMA,
    ],
)
def _sc_gather(idx_hbm, tab_hbm, out_hbm, idx_v, tab_v, out_v, s0, s1, st):
    wid = lax.axis_index("s") * 2 + lax.axis_index("c")
    b = wid // TPB
    chunk = wid % TPB

    def fire(half, sem):
        return [
            pltpu.async_copy(
                idx_hbm.at[pl.ds(
                    (b * L + l) * PLANE + chunk * EPT + half * CH, CH)],
                idx_v.at[pl.ds((half * L + l) * CH, CH)], sem)
            for l in range(L)
        ]

    ct = pltpu.async_copy(tab_hbm.at[pl.ds(b * TABW, TABW)], tab_v, st)
    c0 = fire(0, s0)
    c1 = fire(1, s1)
    for c in c0:
        c.wait()
    ct.wait()

    def compute_half(half):
        base = half * L * CH
        obase = half * CH

        @plsc.parallel_loop(0, CH // 16, unroll=2)
        def _(i):
            g = [
                plsc.load_gather(
                    tab_v.at[pl.ds(l * V, V)],
                    [idx_v[pl.ds(base + l * CH + i * 16, 16)]])
                for l in range(L)
            ]
            out_v[pl.ds(obase + i * 16, 16)] = (
                (g[0] + g[1]) + (g[2] + g[3]) + g[4])

        return pltpu.async_copy(
            out_v.at[pl.ds(obase, CH)],
            out_hbm.at[pl.ds(wid * EPT + obase, CH)], st)

    o0 = compute_half(0)
    for c in c1:
        c.wait()
    o1 = compute_half(1)
    o0.wait()
    o1.wait()


def kernel(edge_embedding, edge_paths, edge_vector):
    emb_t = edge_embedding.transpose(0, 2, 1)
    tab = _make_table(edge_vector.astype(jnp.float32), emb_t)
    idx_t = (edge_paths.astype(jnp.int32)
             .reshape(B, 32, 8, 2, 128, L)
             .transpose(0, 5, 1, 3, 2, 4)
             .reshape(-1))
    out = _sc_gather(idx_t, tab.reshape(-1))
    return (out.reshape(B, 32, 2, 8, 128)
            .transpose(0, 1, 3, 2, 4)
            .reshape(B, N, N))
